# baseline (speedup 1.0000x reference)
import functools

import jax
import jax.numpy as jnp
from jax import lax
from jax.experimental import pallas as pl
from jax.experimental.pallas import tpu as pltpu
from jax.experimental.pallas import tpu_sc as plsc

NUM_EMBEDDINGS = 1000000
EMBEDDING_DIM = 64
BATCH = 4096
SEQ = 200

NC = 2
NS = 16
NW = NC * NS
BB = BATCH // NW
ST = SEQ // 8
DT = EMBEDDING_DIM // 8
N_CHUNKS = SEQ
NBUF = 4
NTB = 2
RPAD = EMBEDDING_DIM + 1


def _make_gather():
    mesh = plsc.VectorSubcoreMesh(core_axis_name="c", subcore_axis_name="s")

    @functools.partial(
        pl.kernel,
        out_type=jax.ShapeDtypeStruct((SEQ, DT, NW, 8, BB), jnp.float32),
        mesh=mesh,
        scratch_types=(
            [pltpu.VMEM((ST, 8, BB), jnp.int32),
             pltpu.VMEM((NBUF, BB, EMBEDDING_DIM), jnp.float32),
             pltpu.VMEM((NTB, BB, RPAD), jnp.float32),
             pltpu.VMEM((NTB, EMBEDDING_DIM, BB), jnp.float32)]
            + [pltpu.SemaphoreType.DMA] * (NBUF + NTB)
        ),
        compiler_params=pltpu.CompilerParams(
            use_tc_tiling_on_sc=False, needs_layout_passes=False,
            disable_bounds_checks=True),
    )
    def gather_kernel(idx_hbm, table_hbm, out_hbm, idx_v, rows_v, rpad_v,
                      trans_v, *sems):
        gsems, osems = sems[:NBUF], sems[NBUF:]
        wid = lax.axis_index("s") * NC + lax.axis_index("c")
        pltpu.sync_copy(idx_hbm.at[wid], idx_v)

        iota16 = lax.iota(jnp.int32, 16)
        ones = jnp.ones((16,), jnp.int32)
        lane_idx = [iota16 + 16 * l0 for l0 in range(BB // 16)]

        def start_gather(j, s):
            pltpu.async_copy(
                table_hbm.at[idx_v.at[j // 8, j % 8]], rows_v.at[s], gsems[s])

        def wait_gather(j, s):
            pltpu.make_async_copy(
                table_hbm.at[idx_v.at[j // 8, j % 8]], rows_v.at[s],
                gsems[s]).wait()

        def transpose(s, t):
            rv = rows_v.at[s]
            pv = rpad_v.at[t]
            tv = trans_v.at[t]

            def cbody(l, carry):
                for k in range(EMBEDDING_DIM // 16):
                    pv[l, pl.dslice(16 * k, 16)] = rv[l, pl.dslice(16 * k, 16)]
                return carry

            lax.fori_loop(0, BB, cbody, 0)

            def dbody(d, col):
                for l0 in range(BB // 16):
                    vals = plsc.load_gather(pv, [lane_idx[l0], col])
                    tv[d, pl.dslice(16 * l0, 16)] = vals
                return col + ones

            lax.fori_loop(0, EMBEDDING_DIM, dbody,
                          jnp.zeros((16,), jnp.int32))

        def start_write(j, t):
            for dt in range(DT):
                pltpu.async_copy(
                    trans_v.at[t, pl.ds(8 * dt, 8)],
                    out_hbm.at[j, dt, wid], osems[t])

        def drain_write(j, t):
            for dt in range(DT):
                pltpu.make_async_copy(
                    trans_v.at[t, pl.ds(8 * dt, 8)],
                    out_hbm.at[j, dt, wid], osems[t]).wait()

        for b in range(NBUF - 1):
            start_gather(b, b)

        for j in range(NBUF):
            wait_gather(j, j % NBUF)
            if j >= NTB:
                drain_write(j - NTB, j % NTB)
            transpose(j % NBUF, j % NTB)
            start_write(j, j % NTB)
            start_gather(j + NBUF - 1, (j + NBUF - 1) % NBUF)

        def body(k, carry):
            j0 = k * NBUF
            for b in range(NBUF):
                j = j0 + b
                wait_gather(j, b)
                drain_write(j - NTB, b % NTB)
                transpose(b, b % NTB)
                start_write(j, b % NTB)
                start_gather(j + NBUF - 1, (b + NBUF - 1) % NBUF)
            return carry

        lax.fori_loop(1, N_CHUNKS // NBUF - 1, body, 0)

        j0 = N_CHUNKS - NBUF
        for b in range(NBUF):
            j = j0 + b
            wait_gather(j, b)
            drain_write(j - NTB, b % NTB)
            transpose(b, b % NTB)
            start_write(j, b % NTB)
            if j + NBUF - 1 < N_CHUNKS:
                start_gather(j + NBUF - 1, (b + NBUF - 1) % NBUF)

        drain_write(N_CHUNKS - 2, (N_CHUNKS - 2) % NTB)
        drain_write(N_CHUNKS - 1, (N_CHUNKS - 1) % NTB)

    return gather_kernel


_gather = _make_gather()


def kernel(input_ids, weight):
    wp = jnp.pad(weight, ((0, 0), (0, 128 - EMBEDDING_DIM))).reshape(
        2 * NUM_EMBEDDINGS, EMBEDDING_DIM)
    ids5 = (
        (input_ids.astype(jnp.int32) * 2)
        .reshape(NW, BB, ST, 8)
        .transpose(0, 2, 3, 1)
    )
    out5 = _gather(ids5, wp)
    return jnp.transpose(out5, (2, 4, 0, 1, 3)).reshape(BATCH, SEQ,
                                                        EMBEDDING_DIM)


# speedup vs baseline: 1.5602x; 1.5602x over previous
import functools

import jax
import jax.numpy as jnp
from jax import lax
from jax.experimental import pallas as pl
from jax.experimental.pallas import tpu as pltpu
from jax.experimental.pallas import tpu_sc as plsc

NUM_EMBEDDINGS = 1000000
EMBEDDING_DIM = 64
BATCH = 4096
SEQ = 200

NC = 2
NS = 16
NW = NC * NS
BB = BATCH // NW
ST = SEQ // 8
DT = EMBEDDING_DIM // 8
N_CHUNKS = SEQ
NBUF = 4
NTB = 2
TPAD = BB + 1


def _make_gather():
    mesh = plsc.VectorSubcoreMesh(core_axis_name="c", subcore_axis_name="s")

    @functools.partial(
        pl.kernel,
        out_type=jax.ShapeDtypeStruct((SEQ, DT, NW, 8, BB), jnp.float32),
        mesh=mesh,
        scratch_types=(
            [pltpu.VMEM((ST, 8, BB), jnp.int32),
             pltpu.VMEM((NBUF, BB, EMBEDDING_DIM), jnp.float32),
             pltpu.VMEM((NTB, EMBEDDING_DIM, TPAD), jnp.float32)]
            + [pltpu.SemaphoreType.DMA] * (NBUF + NTB)
        ),
        compiler_params=pltpu.CompilerParams(
            use_tc_tiling_on_sc=False, needs_layout_passes=False,
            disable_bounds_checks=True),
    )
    def gather_kernel(idx_hbm, table_hbm, out_hbm, idx_v, rows_v, trans_v,
                      *sems):
        gsems, osems = sems[:NBUF], sems[NBUF:]
        wid = lax.axis_index("s") * NC + lax.axis_index("c")
        pltpu.sync_copy(idx_hbm.at[wid], idx_v)

        iota16 = lax.iota(jnp.int32, 16)
        ones = jnp.ones((16,), jnp.int32)

        def start_gather(j, s):
            pltpu.async_copy(
                table_hbm.at[idx_v.at[j // 8, j % 8]], rows_v.at[s], gsems[s])

        def wait_gather(j, s):
            pltpu.make_async_copy(
                table_hbm.at[idx_v.at[j // 8, j % 8]], rows_v.at[s],
                gsems[s]).wait()

        row_idx = [iota16 + 16 * k for k in range(EMBEDDING_DIM // 16)]

        def transpose(s, t):
            rv = rows_v.at[s]
            tv = trans_v.at[t]

            def lbody(li, carry):
                l0 = li * 8
                for u in range(8):
                    col = jnp.full((16,), u, jnp.int32) + l0
                    for k in range(EMBEDDING_DIM // 16):
                        vals = rv[l0 + u, pl.dslice(16 * k, 16)]
                        plsc.store_scatter(tv, [row_idx[k], col], vals)
                return carry

            lax.fori_loop(0, BB // 8, lbody, 0)

        def start_write(j, t):
            for dt in range(DT):
                pltpu.async_copy(
                    trans_v.at[t, pl.ds(8 * dt, 8), pl.dslice(0, BB)],
                    out_hbm.at[j, dt, wid], osems[t])

        def drain_write(j, t):
            for dt in range(DT):
                pltpu.make_async_copy(
                    trans_v.at[t, pl.ds(8 * dt, 8), pl.dslice(0, BB)],
                    out_hbm.at[j, dt, wid], osems[t]).wait()

        for b in range(NBUF - 1):
            start_gather(b, b)

        for j in range(NBUF):
            wait_gather(j, j % NBUF)
            if j >= NTB:
                drain_write(j - NTB, j % NTB)
            transpose(j % NBUF, j % NTB)
            start_write(j, j % NTB)
            start_gather(j + NBUF - 1, (j + NBUF - 1) % NBUF)

        def body(k, carry):
            j0 = k * NBUF
            for b in range(NBUF):
                j = j0 + b
                wait_gather(j, b)
                drain_write(j - NTB, b % NTB)
                transpose(b, b % NTB)
                start_write(j, b % NTB)
                start_gather(j + NBUF - 1, (b + NBUF - 1) % NBUF)
            return carry

        lax.fori_loop(1, N_CHUNKS // NBUF - 1, body, 0)

        j0 = N_CHUNKS - NBUF
        for b in range(NBUF):
            j = j0 + b
            wait_gather(j, b)
            drain_write(j - NTB, b % NTB)
            transpose(b, b % NTB)
            start_write(j, b % NTB)
            if j + NBUF - 1 < N_CHUNKS:
                start_gather(j + NBUF - 1, (b + NBUF - 1) % NBUF)

        drain_write(N_CHUNKS - 2, (N_CHUNKS - 2) % NTB)
        drain_write(N_CHUNKS - 1, (N_CHUNKS - 1) % NTB)

    return gather_kernel


_gather = _make_gather()


def kernel(input_ids, weight):
    wp = jnp.pad(weight, ((0, 0), (0, 128 - EMBEDDING_DIM))).reshape(
        2 * NUM_EMBEDDINGS, EMBEDDING_DIM)
    ids5 = (
        (input_ids.astype(jnp.int32) * 2)
        .reshape(NW, BB, ST, 8)
        .transpose(0, 2, 3, 1)
    )
    out5 = _gather(ids5, wp)
    return jnp.transpose(out5, (2, 4, 0, 1, 3)).reshape(BATCH, SEQ,
                                                        EMBEDDING_DIM)
